# baseline (speedup 1.0000x reference)
import functools

import jax
import jax.numpy as jnp
import numpy as np
from jax import lax
from jax.experimental import pallas as pl
from jax.experimental.pallas import tpu as pltpu
from jax.experimental.pallas import tpu_sc as plsc

N = 10000
E = 160000
D = 512
PN = 10240

NC, NS, L = 2, 16, 16
NW = NC * NS
EW = E // NW
B = 32
EWPAD = 5120
NBATCH = EWPAD // B
SCHUNK = 2048
SPANS = (SCHUNK,) * (PN // SCHUNK)
ACC_ROWS = SCHUNK + NS

_mesh = plsc.VectorSubcoreMesh(core_axis_name="c", subcore_axis_name="s")

f32 = jnp.float32
i32 = jnp.int32


def _zero16f():
    return jnp.zeros((L,), f32)


def _zero16i():
    return jnp.zeros((L,), i32)


def _deg_body(col_hbm, ew_hbm, d1_hbm, d2_hbm,
              colb, ewb, idxb, idxt, valt, onesb, zb, d1acc, d2acc):
    c = lax.axis_index("c")
    s = lax.axis_index("s")
    wid = c * NS + s
    e0 = wid * EW
    pltpu.sync_copy(col_hbm.at[pl.ds(e0, EW)], colb.at[pl.ds(0, EW)])
    pltpu.sync_copy(ew_hbm.at[pl.ds(e0, EW)], ewb.at[pl.ds(0, EW)])

    def _fill(i, _):
        off = pl.multiple_of(i * L, L)
        zb[pl.ds(off, L)] = _zero16f()
        return 0
    lax.fori_loop(0, 640 // L, _fill, 0)
    for g in range(128 // L):
        onesb[pl.ds(g * L, L)] = jnp.ones((L,), f32)

    pltpu.sync_copy(zb, d1acc.at[pl.ds(s * 640, 640)])
    pltpu.sync_copy(zb, d2acc.at[pl.ds(s * 640, 640)])
    plsc.subcore_barrier()

    def _batch(k, _):
        off = pl.multiple_of(k * 128, 128)
        for g in range(128 // L):
            idxb[pl.ds(g * L, L)] = colb[pl.ds(off + g * L, L)]
        pltpu.sync_copy(ewb.at[pl.ds(off, 128)], d1acc.at[idxb], add=True)
        pltpu.sync_copy(onesb, d2acc.at[idxb], add=True)
        return 0
    lax.fori_loop(0, EW // 128, _batch, 0)

    toff = (EW // 128) * 128
    valid = lax.iota(i32, L) < (EW - toff)
    col16 = colb[pl.ds(toff, L)]
    ew16 = ewb[pl.ds(toff, L)]
    idxt[pl.ds(0, L)] = jnp.where(valid, col16, 0)
    valt[pl.ds(0, L)] = jnp.where(valid, ew16, 0.0)
    pltpu.sync_copy(valt, d1acc.at[idxt], add=True)
    valt[pl.ds(0, L)] = jnp.where(valid, jnp.ones((L,), f32), 0.0)
    pltpu.sync_copy(valt, d2acc.at[idxt], add=True)

    plsc.subcore_barrier()

    @pl.when(s == 0)
    def _():
        pltpu.sync_copy(d1acc, d1_hbm.at[c])
        pltpu.sync_copy(d2acc, d2_hbm.at[c])


def _degrees(col, ew):
    return pl.kernel(
        _deg_body,
        out_type=[jax.ShapeDtypeStruct((NC, PN), f32),
                  jax.ShapeDtypeStruct((NC, PN), f32)],
        mesh=_mesh,
        scratch_types=[
            pltpu.VMEM((EWPAD,), i32),
            pltpu.VMEM((EWPAD,), f32),
            pltpu.VMEM((128,), i32),
            pltpu.VMEM((L,), i32),
            pltpu.VMEM((L,), f32),
            pltpu.VMEM((128,), f32),
            pltpu.VMEM((640,), f32),
            pltpu.VMEM_SHARED((PN,), f32),
            pltpu.VMEM_SHARED((PN,), f32),
        ],
    )(col, ew)


def _splat_lane(v, lane):
    idx = jnp.zeros((L,), i32) + lane
    return lax.gather(
        v, idx[:, None],
        dimension_numbers=lax.GatherDimensionNumbers(
            offset_dims=(), collapsed_slice_dims=(0,), start_index_map=(0,)),
        slice_sizes=(1,),
        mode=lax.GatherScatterMode.PROMISE_IN_BOUNDS)


DW = 128
XP = D // DW


def _spmm_body(has_ew, *refs):
    if has_ew:
        (y_hbm, row_hbm, col_hbm, ew_hbm, parts_hbm,
         rowb, colb, ewb, ridx, cidx, ridx2, cidx2, gbuf, gbuf2, zb, acc,
         gsem, ssem, ssem2) = refs
    else:
        (y_hbm, row_hbm, col_hbm, parts_hbm,
         rowb, colb, ridx, cidx, ridx2, cidx2, gbuf, gbuf2, zb, acc,
         gsem, ssem, ssem2) = refs

    c = lax.axis_index("c")
    s = lax.axis_index("s")
    wid = c * NS + s
    e0 = wid * EW
    pltpu.sync_copy(row_hbm.at[pl.ds(e0, EW)], rowb.at[pl.ds(0, EW)])
    pltpu.sync_copy(col_hbm.at[pl.ds(e0, EW)], colb.at[pl.ds(0, EW)])
    if has_ew:
        pltpu.sync_copy(ew_hbm.at[pl.ds(e0, EW)], ewb.at[pl.ds(0, EW)])

    def _zrow(i, _):
        for u in range(XP):
            for j in range(DW // L):
                zb[i, u, pl.ds(j * L, L)] = _zero16f()
        return 0
    lax.fori_loop(0, 8, _zrow, 0)

    lanes = lax.iota(i32, L)
    dumpv = jnp.zeros((L,), i32) + (SCHUNK + s)

    for ci, span in enumerate(SPANS):
        lo = ci * SCHUNK
        rpw = span // NS

        for t in range(rpw // 8):
            pltpu.sync_copy(zb, acc.at[pl.ds(s * rpw + t * 8, 8)])
        plsc.subcore_barrier()

        def _batch2(mi, _):
            for half, (gb, rix, cix, ss) in enumerate(
                    ((gbuf, ridx, cidx, ssem), (gbuf2, ridx2, cidx2, ssem2))):
                off = pl.multiple_of((mi * 2 + half) * B, B)

                @pl.when(mi > 0)
                def _():
                    pltpu.make_async_copy(gb, acc.at[cix], ss).wait()

                for g in range(B // L):
                    col16 = colb[pl.ds(off + g * L, L)]
                    row16 = rowb[pl.ds(off + g * L, L)]
                    valid = (lanes + (off + g * L)) < EW
                    m = valid & (col16 >= lo) & (col16 < lo + SCHUNK)
                    rix[pl.ds(g * L, L)] = jnp.where(valid, row16, 0)
                    cix[pl.ds(g * L, L)] = jnp.where(m, col16 - lo, dumpv)
                pltpu.async_copy(y_hbm.at[rix], gb, gsem).wait()
                if has_ew:
                    def _srow(e, _):
                        o2 = pl.multiple_of(off + (e // L) * L, L)
                        ew16 = ewb[pl.ds(o2, L)]
                        spl = _splat_lane(ew16, e % L)
                        for u in range(XP):
                            for j in range(DW // L):
                                gb[e, u, pl.ds(j * L, L)] = \
                                    gb[e, u, pl.ds(j * L, L)] * spl
                        return 0
                    lax.fori_loop(0, B, _srow, 0)
                pltpu.async_copy(gb, acc.at[cix], ss, add=True)
            return 0
        lax.fori_loop(0, NBATCH // 2, _batch2, 0)
        pltpu.make_async_copy(gbuf, acc.at[cidx], ssem).wait()
        pltpu.make_async_copy(gbuf2, acc.at[cidx2], ssem2).wait()
        plsc.subcore_barrier()

        for t in range(rpw // 16):
            r0 = s * rpw + t * 16
            pltpu.sync_copy(acc.at[pl.ds(r0, 16)],
                            parts_hbm.at[c, pl.ds(lo + r0, 16)])
        plsc.subcore_barrier()


def _spmm(y, row, col, ew=None):
    has_ew = ew is not None
    scratch = [
        pltpu.VMEM((EWPAD,), i32),
        pltpu.VMEM((EWPAD,), i32),
    ]
    if has_ew:
        scratch.append(pltpu.VMEM((EWPAD,), f32))
    scratch += [
        pltpu.VMEM((B,), i32),
        pltpu.VMEM((B,), i32),
        pltpu.VMEM((B,), i32),
        pltpu.VMEM((B,), i32),
        pltpu.VMEM((B, XP, DW), f32),
        pltpu.VMEM((B, XP, DW), f32),
        pltpu.VMEM((8, XP, DW), f32),
        pltpu.VMEM_SHARED((ACC_ROWS, XP, DW), f32),
        pltpu.SemaphoreType.DMA,
        pltpu.SemaphoreType.DMA,
        pltpu.SemaphoreType.DMA,
    ]
    args = (y.reshape(N, XP, DW), row, col)
    if has_ew:
        args = args + (ew,)
    parts4 = pl.kernel(
        functools.partial(_spmm_body, has_ew),
        out_type=jax.ShapeDtypeStruct((NC, PN, XP, DW), f32),
        mesh=_mesh,
        scratch_types=scratch,
    )(*args)
    return parts4.reshape(NC, PN, D)


def _dinv_body(d1_ref, d2_ref, o1_ref, o2_ref):
    o1_ref[...] = lax.rsqrt(d1_ref[0] + d1_ref[1] + 1.0)
    o2_ref[...] = lax.rsqrt(d2_ref[0] + d2_ref[1] + 1.0)


def _dinvs(d1p, d2p):
    return pl.pallas_call(
        _dinv_body,
        out_shape=[jax.ShapeDtypeStruct((PN,), f32),
                   jax.ShapeDtypeStruct((PN,), f32)],
    )(d1p, d2p)


BM = 1000


def _mm_scale_body(x_ref, w_ref, dv_ref, y_ref):
    y_ref[...] = jnp.dot(x_ref[...], w_ref[...],
                         preferred_element_type=f32) * dv_ref[...]


def _mm_scale(x, w, dv):
    return pl.pallas_call(
        _mm_scale_body,
        grid=(N // BM,),
        in_specs=[
            pl.BlockSpec((BM, D), lambda i: (i, 0)),
            pl.BlockSpec((D, D), lambda i: (0, 0)),
            pl.BlockSpec((BM, 1), lambda i: (i, 0)),
        ],
        out_specs=pl.BlockSpec((BM, D), lambda i: (i, 0)),
        out_shape=jax.ShapeDtypeStruct((N, D), f32),
    )(x, w, dv)


def _mid_body(p_ref, y1_ref, dv1_ref, b1_ref, w2_ref, dv2_ref, y2_ref):
    pre = (p_ref[0] + p_ref[1] + y1_ref[...]) * dv1_ref[...] + b1_ref[...]
    h = jnp.maximum(pre, 0.0)
    y2_ref[...] = jnp.dot(h, w2_ref[...], preferred_element_type=f32) * dv2_ref[...]


def _mid(p1, y1, dv1, b1, w2, dv2):
    return pl.pallas_call(
        _mid_body,
        grid=(N // BM,),
        in_specs=[
            pl.BlockSpec((NC, BM, D), lambda i: (0, i, 0)),
            pl.BlockSpec((BM, D), lambda i: (i, 0)),
            pl.BlockSpec((BM, 1), lambda i: (i, 0)),
            pl.BlockSpec((1, D), lambda i: (0, 0)),
            pl.BlockSpec((D, D), lambda i: (0, 0)),
            pl.BlockSpec((BM, 1), lambda i: (i, 0)),
        ],
        out_specs=pl.BlockSpec((BM, D), lambda i: (i, 0)),
        out_shape=jax.ShapeDtypeStruct((N, D), f32),
    )(p1, y1, dv1, b1, w2, dv2)


def _final_body(p_ref, y2_ref, dv2_ref, b2_ref, o_ref):
    o_ref[...] = (p_ref[0] + p_ref[1] + y2_ref[...]) * dv2_ref[...] + b2_ref[...]


def _final(p2, y2, dv2, b2):
    return pl.pallas_call(
        _final_body,
        grid=(N // BM,),
        in_specs=[
            pl.BlockSpec((NC, BM, D), lambda i: (0, i, 0)),
            pl.BlockSpec((BM, D), lambda i: (i, 0)),
            pl.BlockSpec((BM, 1), lambda i: (i, 0)),
            pl.BlockSpec((1, D), lambda i: (0, 0)),
        ],
        out_specs=pl.BlockSpec((BM, D), lambda i: (i, 0)),
        out_shape=jax.ShapeDtypeStruct((N, D), f32),
    )(p2, y2, dv2, b2)


@jax.jit
def kernel(x, edge_index, edge_attr, W1, b1, W2, b2):
    row = edge_index[0]
    col = edge_index[1]

    d1p, d2p = _degrees(col, edge_attr)
    dinv1, dinv2 = _dinvs(d1p, d2p)
    dv1 = dinv1[:N].reshape(N, 1)
    dv2 = dinv2[:N].reshape(N, 1)

    y1 = _mm_scale(x, W1, dv1)
    p1 = _spmm(y1, row, col, edge_attr)
    y2 = _mid(p1, y1, dv1, b1.reshape(1, D), W2, dv2)
    p2 = _spmm(y2, row, col)
    return _final(p2, y2, dv2, b2.reshape(1, D))


# speedup vs baseline: 2.3548x; 2.3548x over previous
import functools

import jax
import jax.numpy as jnp
import numpy as np
from jax import lax
from jax.experimental import pallas as pl
from jax.experimental.pallas import tpu as pltpu
from jax.experimental.pallas import tpu_sc as plsc

N = 10000
E = 160000
D = 512
PN = 10240

NC, NS, L = 2, 16, 16
NW = NC * NS
EW = E // NW
B = 64
EWPAD = 5120
NBATCH = EWPAD // B
SCHUNK = 2048
NCH = PN // SCHUNK
SPANS = (SCHUNK,) * NCH
ACC_ROWS = SCHUNK + NS

EP = EWPAD * NW
ES = E + NCH * B * 2
ES2 = ES + (EP - E)
ER = E // 128

def _mesh():
    return plsc.VectorSubcoreMesh(core_axis_name="c", subcore_axis_name="s")

f32 = jnp.float32
i32 = jnp.int32


def _zero16f():
    return jnp.zeros((L,), f32)


def _zero16i():
    return jnp.zeros((L,), i32)


def _deg_body(col_hbm, ew_hbm, d1_hbm, d2_hbm,
              colb, ewb, idxb, idxt, valt, onesb, zb, d1acc, d2acc):
    c = lax.axis_index("c")
    s = lax.axis_index("s")
    wid = c * NS + s
    e0 = wid * EW
    pltpu.sync_copy(col_hbm.at[pl.ds(e0, EW)], colb.at[pl.ds(0, EW)])
    pltpu.sync_copy(ew_hbm.at[pl.ds(e0, EW)], ewb.at[pl.ds(0, EW)])

    def _fill(i, _):
        off = pl.multiple_of(i * L, L)
        zb[pl.ds(off, L)] = _zero16f()
        return 0
    lax.fori_loop(0, 640 // L, _fill, 0)
    for g in range(128 // L):
        onesb[pl.ds(g * L, L)] = jnp.ones((L,), f32)

    pltpu.sync_copy(zb, d1acc.at[pl.ds(s * 640, 640)])
    pltpu.sync_copy(zb, d2acc.at[pl.ds(s * 640, 640)])
    plsc.subcore_barrier()

    def _batch(k, _):
        off = pl.multiple_of(k * 128, 128)
        for g in range(128 // L):
            idxb[pl.ds(g * L, L)] = colb[pl.ds(off + g * L, L)]
        pltpu.sync_copy(ewb.at[pl.ds(off, 128)], d1acc.at[idxb], add=True)
        pltpu.sync_copy(onesb, d2acc.at[idxb], add=True)
        return 0
    lax.fori_loop(0, EW // 128, _batch, 0)

    toff = (EW // 128) * 128
    valid = lax.iota(i32, L) < (EW - toff)
    col16 = colb[pl.ds(toff, L)]
    ew16 = ewb[pl.ds(toff, L)]
    idxt[pl.ds(0, L)] = jnp.where(valid, col16, 0)
    valt[pl.ds(0, L)] = jnp.where(valid, ew16, 0.0)
    pltpu.sync_copy(valt, d1acc.at[idxt], add=True)
    valt[pl.ds(0, L)] = jnp.where(valid, jnp.ones((L,), f32), 0.0)
    pltpu.sync_copy(valt, d2acc.at[idxt], add=True)

    plsc.subcore_barrier()

    @pl.when(s == 0)
    def _():
        pltpu.sync_copy(d1acc, d1_hbm.at[c])
        pltpu.sync_copy(d2acc, d2_hbm.at[c])


def _degrees(col, ew):
    return pl.kernel(
        _deg_body,
        out_type=[jax.ShapeDtypeStruct((NC, PN), f32),
                  jax.ShapeDtypeStruct((NC, PN), f32)],
        mesh=_mesh(),
        scratch_types=[
            pltpu.VMEM((EWPAD,), i32),
            pltpu.VMEM((EWPAD,), f32),
            pltpu.VMEM((128,), i32),
            pltpu.VMEM((L,), i32),
            pltpu.VMEM((L,), f32),
            pltpu.VMEM((128,), f32),
            pltpu.VMEM((640,), f32),
            pltpu.VMEM_SHARED((PN,), f32),
            pltpu.VMEM_SHARED((PN,), f32),
        ],
    )(col, ew)


def _bucket_body(col_ref, pos_ref, binfo_ref):
    cols = col_ref[...]
    q = lax.shift_right_logical(cols, 11)
    jrow = lax.broadcasted_iota(i32, (128, 128), 0)
    jcol = lax.broadcasted_iota(i32, (128, 128), 1)
    su128 = (jrow < jcol).astype(f32)
    irow = lax.broadcasted_iota(i32, (ER, ER), 0)
    icol = lax.broadcasted_iota(i32, (ER, ER), 1)
    suer = (irow < icol).astype(f32)

    pos = jnp.zeros(cols.shape, f32)
    start = 0.0
    starts, tots, nbs = [], [], []
    for qq in range(NCH):
        m = (q == qq).astype(f32)
        within = jnp.dot(m, su128, preferred_element_type=f32)
        rs = jnp.sum(m, axis=1)
        rp = jnp.dot(rs.reshape(1, ER), suer,
                     preferred_element_type=f32)
        posq = rp.reshape(ER, 1) + within
        tot = jnp.sum(rs)
        pos = pos + m * (posq + start)
        starts.append(start)
        tots.append(tot)
        nbs.append(jnp.ceil(tot / B))
        start = start + jnp.ceil(tot / B) * B
    pos_ref[...] = pos.astype(i32)
    lane = lax.broadcasted_iota(i32, (128,), 0)
    info = jnp.zeros((128,), f32)
    for k, v in enumerate(starts + tots + nbs):
        info = info + jnp.where(lane == k, v, 0.0)
    binfo_ref[...] = info.astype(i32)


def _bucket(col2d):
    return pl.pallas_call(
        _bucket_body,
        out_shape=[jax.ShapeDtypeStruct((ER, 128), i32),
                   jax.ShapeDtypeStruct((128,), i32)],
    )(col2d)


def _permute_body(row_hbm, col_hbm, ew_hbm, pos_hbm,
                  rows_hbm, cols_hbm, ews_hbm,
                  rowb, colb, ewb, posb, pidx, sem, sem2, sem3):
    c = lax.axis_index("c")
    s = lax.axis_index("s")
    wid = c * NS + s
    e0 = wid * EWPAD
    pltpu.sync_copy(row_hbm.at[pl.ds(e0, EWPAD)], rowb)
    pltpu.sync_copy(col_hbm.at[pl.ds(e0, EWPAD)], colb)
    pltpu.sync_copy(ew_hbm.at[pl.ds(e0, EWPAD)], ewb)
    pltpu.sync_copy(pos_hbm.at[pl.ds(e0, EWPAD)], posb)

    def _bt(k, _):
        off = pl.multiple_of(k * 128, 128)
        for g in range(128 // L):
            pidx[pl.ds(g * L, L)] = posb[pl.ds(off + g * L, L)]
        d1 = pltpu.async_copy(rowb.at[pl.ds(off, 128)], rows_hbm.at[pidx], sem)
        d2 = pltpu.async_copy(colb.at[pl.ds(off, 128)], cols_hbm.at[pidx], sem2)
        d3 = pltpu.async_copy(ewb.at[pl.ds(off, 128)], ews_hbm.at[pidx], sem3)
        d1.wait()
        d2.wait()
        d3.wait()
        return 0
    lax.fori_loop(0, EWPAD // 128, _bt, 0)


def _permute(row_p, col_p, ew_p, pos_p):
    return pl.kernel(
        _permute_body,
        out_type=[jax.ShapeDtypeStruct((ES2,), i32),
                  jax.ShapeDtypeStruct((ES2,), i32),
                  jax.ShapeDtypeStruct((ES2,), f32)],
        mesh=_mesh(),
        scratch_types=[
            pltpu.VMEM((EWPAD,), i32),
            pltpu.VMEM((EWPAD,), i32),
            pltpu.VMEM((EWPAD,), f32),
            pltpu.VMEM((EWPAD,), i32),
            pltpu.VMEM((128,), i32),
            pltpu.SemaphoreType.DMA,
            pltpu.SemaphoreType.DMA,
            pltpu.SemaphoreType.DMA,
        ],
    )(row_p, col_p, ew_p, pos_p)


def _splat_lane(v, lane):
    idx = jnp.zeros((L,), i32) + lane
    return lax.gather(
        v, idx[:, None],
        dimension_numbers=lax.GatherDimensionNumbers(
            offset_dims=(), collapsed_slice_dims=(0,), start_index_map=(0,)),
        slice_sizes=(1,),
        mode=lax.GatherScatterMode.PROMISE_IN_BOUNDS)


DW = 128
XP = D // DW


def _spmm_body(has_ew, *refs):
    if has_ew:
        (y_hbm, rows_hbm, cols_hbm, binfo_hbm, ews_hbm, parts_hbm,
         rsb, csb, esb, binb, ridx, cidx, gbuf, zb, acc, gsem) = refs
    else:
        (y_hbm, rows_hbm, cols_hbm, binfo_hbm, parts_hbm,
         rsb, csb, binb, ridx, cidx, gbuf, zb, acc, gsem) = refs

    c = lax.axis_index("c")
    s = lax.axis_index("s")
    wid = c * NS + s
    pltpu.sync_copy(binfo_hbm.at[pl.ds(0, L)], binb)
    bvec = binb[pl.ds(0, L)]

    def _zrow(i, _):
        for u in range(XP):
            for j in range(DW // L):
                zb[i, u, pl.ds(j * L, L)] = _zero16f()
        return 0
    lax.fori_loop(0, 8, _zrow, 0)

    lanes = lax.iota(i32, L)
    dumpv = jnp.zeros((L,), i32) + (SCHUNK + s)

    for ci, span in enumerate(SPANS):
        lo = ci * SCHUNK
        rpw = span // NS
        st = bvec[ci]
        tot = bvec[NCH + ci]
        nbb = bvec[2 * NCH + ci]

        for t in range(rpw // 8):
            pltpu.sync_copy(zb, acc.at[pl.ds(s * rpw + t * 8, 8)])
        plsc.subcore_barrier()

        cnt = (nbb - wid + (NW - 1)) // NW

        def _batch(j, _):
            goff = pl.multiple_of(st + (wid + j * NW) * B, B)
            pltpu.sync_copy(rows_hbm.at[pl.ds(goff, B)], rsb)
            pltpu.sync_copy(cols_hbm.at[pl.ds(goff, B)], csb)
            if has_ew:
                pltpu.sync_copy(ews_hbm.at[pl.ds(goff, B)], esb)
            for g in range(B // L):
                col16 = csb[pl.ds(g * L, L)]
                row16 = rsb[pl.ds(g * L, L)]
                valid = (lanes + (goff + g * L)) < (st + tot)
                m = valid & (col16 >= lo) & (col16 < lo + SCHUNK)
                ridx[pl.ds(g * L, L)] = jnp.where(valid, row16, 0)
                cidx[pl.ds(g * L, L)] = jnp.where(m, col16 - lo, dumpv)
            pltpu.async_copy(y_hbm.at[ridx], gbuf, gsem).wait()
            if has_ew:
                def _srow(e, _):
                    o2 = pl.multiple_of((e // L) * L, L)
                    ew16 = esb[pl.ds(o2, L)]
                    spl = _splat_lane(ew16, e % L)
                    for u in range(XP):
                        for j2 in range(DW // L):
                            gbuf[e, u, pl.ds(j2 * L, L)] = \
                                gbuf[e, u, pl.ds(j2 * L, L)] * spl
                    return 0
                lax.fori_loop(0, B, _srow, 0)
            pltpu.sync_copy(gbuf, acc.at[cidx], add=True)
            return 0
        lax.fori_loop(0, cnt, _batch, 0)
        plsc.subcore_barrier()

        for t in range(rpw // 16):
            r0 = s * rpw + t * 16
            pltpu.sync_copy(acc.at[pl.ds(r0, 16)],
                            parts_hbm.at[c, pl.ds(lo + r0, 16)])
        plsc.subcore_barrier()


def _spmm(y, rows_s, cols_s, binfo, ews_s=None):
    has_ew = ews_s is not None
    scratch = [
        pltpu.VMEM((B,), i32),
        pltpu.VMEM((B,), i32),
    ]
    if has_ew:
        scratch.append(pltpu.VMEM((B,), f32))
    scratch += [
        pltpu.VMEM((L,), i32),
        pltpu.VMEM((B,), i32),
        pltpu.VMEM((B,), i32),
        pltpu.VMEM((B, XP, DW), f32),
        pltpu.VMEM((8, XP, DW), f32),
        pltpu.VMEM_SHARED((ACC_ROWS, XP, DW), f32),
        pltpu.SemaphoreType.DMA,
    ]
    args = (y.reshape(N, XP, DW), rows_s, cols_s, binfo)
    if has_ew:
        args = args + (ews_s,)
    parts4 = pl.kernel(
        functools.partial(_spmm_body, has_ew),
        out_type=jax.ShapeDtypeStruct((NC, PN, XP, DW), f32),
        mesh=_mesh(),
        scratch_types=scratch,
    )(*args)
    return parts4.reshape(NC, PN, D)


def _dinv_body(d1_ref, d2_ref, o1_ref, o2_ref):
    o1_ref[...] = lax.rsqrt(d1_ref[0] + d1_ref[1] + 1.0)
    o2_ref[...] = lax.rsqrt(d2_ref[0] + d2_ref[1] + 1.0)


def _dinvs(d1p, d2p):
    return pl.pallas_call(
        _dinv_body,
        out_shape=[jax.ShapeDtypeStruct((PN,), f32),
                   jax.ShapeDtypeStruct((PN,), f32)],
    )(d1p, d2p)


BM = 1000


def _mm_scale_body(x_ref, w_ref, dv_ref, y_ref):
    y_ref[...] = jnp.dot(x_ref[...], w_ref[...],
                         preferred_element_type=f32) * dv_ref[...]


def _mm_scale(x, w, dv):
    return pl.pallas_call(
        _mm_scale_body,
        grid=(N // BM,),
        in_specs=[
            pl.BlockSpec((BM, D), lambda i: (i, 0)),
            pl.BlockSpec((D, D), lambda i: (0, 0)),
            pl.BlockSpec((BM, 1), lambda i: (i, 0)),
        ],
        out_specs=pl.BlockSpec((BM, D), lambda i: (i, 0)),
        out_shape=jax.ShapeDtypeStruct((N, D), f32),
    )(x, w, dv)


def _mid_body(p_ref, y1_ref, dv1_ref, b1_ref, w2_ref, dv2_ref, y2_ref):
    pre = (p_ref[0] + p_ref[1] + y1_ref[...]) * dv1_ref[...] + b1_ref[...]
    h = jnp.maximum(pre, 0.0)
    y2_ref[...] = jnp.dot(h, w2_ref[...], preferred_element_type=f32) * dv2_ref[...]


def _mid(p1, y1, dv1, b1, w2, dv2):
    return pl.pallas_call(
        _mid_body,
        grid=(N // BM,),
        in_specs=[
            pl.BlockSpec((NC, BM, D), lambda i: (0, i, 0)),
            pl.BlockSpec((BM, D), lambda i: (i, 0)),
            pl.BlockSpec((BM, 1), lambda i: (i, 0)),
            pl.BlockSpec((1, D), lambda i: (0, 0)),
            pl.BlockSpec((D, D), lambda i: (0, 0)),
            pl.BlockSpec((BM, 1), lambda i: (i, 0)),
        ],
        out_specs=pl.BlockSpec((BM, D), lambda i: (i, 0)),
        out_shape=jax.ShapeDtypeStruct((N, D), f32),
    )(p1, y1, dv1, b1, w2, dv2)


def _final_body(p_ref, y2_ref, dv2_ref, b2_ref, o_ref):
    o_ref[...] = (p_ref[0] + p_ref[1] + y2_ref[...]) * dv2_ref[...] + b2_ref[...]


def _final(p2, y2, dv2, b2):
    return pl.pallas_call(
        _final_body,
        grid=(N // BM,),
        in_specs=[
            pl.BlockSpec((NC, BM, D), lambda i: (0, i, 0)),
            pl.BlockSpec((BM, D), lambda i: (i, 0)),
            pl.BlockSpec((BM, 1), lambda i: (i, 0)),
            pl.BlockSpec((1, D), lambda i: (0, 0)),
        ],
        out_specs=pl.BlockSpec((BM, D), lambda i: (i, 0)),
        out_shape=jax.ShapeDtypeStruct((N, D), f32),
    )(p2, y2, dv2, b2)


@jax.jit
def kernel(x, edge_index, edge_attr, W1, b1, W2, b2):
    row = edge_index[0]
    col = edge_index[1]

    d1p, d2p = _degrees(col, edge_attr)
    dinv1, dinv2 = _dinvs(d1p, d2p)
    dv1 = dinv1[:N].reshape(N, 1)
    dv2 = dinv2[:N].reshape(N, 1)

    pos2d, binfo = _bucket(col.reshape(ER, 128))
    npad = EP - E
    pos_p = jnp.concatenate([pos2d.reshape(E),
                             ES + jnp.arange(npad, dtype=i32)])
    row_p = jnp.concatenate([row, jnp.zeros((npad,), i32)])
    col_p = jnp.concatenate([col, jnp.full((npad,), PN, i32)])
    ew_p = jnp.concatenate([edge_attr, jnp.zeros((npad,), f32)])
    rows_s, cols_s, ews_s = _permute(row_p, col_p, ew_p, pos_p)

    y1 = _mm_scale(x, W1, dv1)
    p1 = _spmm(y1, rows_s, cols_s, binfo, ews_s)
    y2 = _mid(p1, y1, dv1, b1.reshape(1, D), W2, dv2)
    p2 = _spmm(y2, rows_s, cols_s, binfo)
    return _final(p2, y2, dv2, b2.reshape(1, D))


# speedup vs baseline: 3.7559x; 1.5950x over previous
import functools

import jax
import jax.numpy as jnp
import numpy as np
from jax import lax
from jax.experimental import pallas as pl
from jax.experimental.pallas import tpu as pltpu
from jax.experimental.pallas import tpu_sc as plsc

N = 10000
E = 160000
D = 512
PN = 10240

NC, NS, L = 2, 16, 16
NW = NC * NS
EW = E // NW
B = 64
EWPAD = 5120
NBATCH = EWPAD // B
SCHUNK = 2048
NCH = PN // SCHUNK
SPANS = (SCHUNK,) * NCH
ACC_ROWS = SCHUNK + NS

EP = EWPAD * NW
ES = E + NCH * B * 2
ES2 = -(-(ES + (EP - E)) // 2048) * 2048
ER = E // 128

def _mesh():
    return plsc.VectorSubcoreMesh(core_axis_name="c", subcore_axis_name="s")

f32 = jnp.float32
i32 = jnp.int32


def _zero16f():
    return jnp.zeros((L,), f32)


def _zero16i():
    return jnp.zeros((L,), i32)


def _deg_body(col_hbm, ew_hbm, d1_hbm, d2_hbm,
              colb, ewb, idxb, idxt, valt, onesb, zb, d1acc, d2acc):
    c = lax.axis_index("c")
    s = lax.axis_index("s")
    wid = c * NS + s
    e0 = wid * EW
    pltpu.sync_copy(col_hbm.at[pl.ds(e0, EW)], colb.at[pl.ds(0, EW)])
    pltpu.sync_copy(ew_hbm.at[pl.ds(e0, EW)], ewb.at[pl.ds(0, EW)])

    def _fill(i, _):
        off = pl.multiple_of(i * L, L)
        zb[pl.ds(off, L)] = _zero16f()
        return 0
    lax.fori_loop(0, 640 // L, _fill, 0)
    for g in range(128 // L):
        onesb[pl.ds(g * L, L)] = jnp.ones((L,), f32)

    pltpu.sync_copy(zb, d1acc.at[pl.ds(s * 640, 640)])
    pltpu.sync_copy(zb, d2acc.at[pl.ds(s * 640, 640)])
    plsc.subcore_barrier()

    def _batch(k, _):
        off = pl.multiple_of(k * 128, 128)
        for g in range(128 // L):
            idxb[pl.ds(g * L, L)] = colb[pl.ds(off + g * L, L)]
        pltpu.sync_copy(ewb.at[pl.ds(off, 128)], d1acc.at[idxb], add=True)
        pltpu.sync_copy(onesb, d2acc.at[idxb], add=True)
        return 0
    lax.fori_loop(0, EW // 128, _batch, 0)

    toff = (EW // 128) * 128
    valid = lax.iota(i32, L) < (EW - toff)
    col16 = colb[pl.ds(toff, L)]
    ew16 = ewb[pl.ds(toff, L)]
    idxt[pl.ds(0, L)] = jnp.where(valid, col16, 0)
    valt[pl.ds(0, L)] = jnp.where(valid, ew16, 0.0)
    pltpu.sync_copy(valt, d1acc.at[idxt], add=True)
    valt[pl.ds(0, L)] = jnp.where(valid, jnp.ones((L,), f32), 0.0)
    pltpu.sync_copy(valt, d2acc.at[idxt], add=True)

    plsc.subcore_barrier()

    @pl.when(s == 0)
    def _():
        pltpu.sync_copy(d1acc, d1_hbm.at[c])
        pltpu.sync_copy(d2acc, d2_hbm.at[c])


def _degrees(col, ew):
    return pl.kernel(
        _deg_body,
        out_type=[jax.ShapeDtypeStruct((NC, PN), f32),
                  jax.ShapeDtypeStruct((NC, PN), f32)],
        mesh=_mesh(),
        scratch_types=[
            pltpu.VMEM((EWPAD,), i32),
            pltpu.VMEM((EWPAD,), f32),
            pltpu.VMEM((128,), i32),
            pltpu.VMEM((L,), i32),
            pltpu.VMEM((L,), f32),
            pltpu.VMEM((128,), f32),
            pltpu.VMEM((640,), f32),
            pltpu.VMEM_SHARED((PN,), f32),
            pltpu.VMEM_SHARED((PN,), f32),
        ],
    )(col, ew)


EWQ = 131072


def _bucket_body(col_ref, row_ref, ew_ref, pos_ref, binfo_ref, rw_ref):
    cols = col_ref[...]
    rows = row_ref[...]
    ews = ew_ref[...]
    rw_ref[...] = rows * EWQ + jnp.floor(ews * EWQ).astype(i32)
    q = lax.shift_right_logical(cols, 11)
    jrow = lax.broadcasted_iota(i32, (128, 128), 0)
    jcol = lax.broadcasted_iota(i32, (128, 128), 1)
    su128 = (jrow < jcol).astype(f32)
    irow = lax.broadcasted_iota(i32, (ER, ER), 0)
    icol = lax.broadcasted_iota(i32, (ER, ER), 1)
    suer = (irow < icol).astype(f32)

    pos = jnp.zeros(cols.shape, f32)
    start = 0.0
    starts, tots, nbs = [], [], []
    for qq in range(NCH):
        m = (q == qq).astype(f32)
        within = jnp.dot(m, su128, preferred_element_type=f32)
        rs = jnp.sum(m, axis=1)
        rp = jnp.dot(rs.reshape(1, ER), suer,
                     preferred_element_type=f32)
        posq = rp.reshape(ER, 1) + within
        tot = jnp.sum(rs)
        pos = pos + m * (posq + start)
        starts.append(start)
        tots.append(tot)
        nbs.append(jnp.ceil(tot / B))
        start = start + jnp.ceil(tot / B) * B
    pos_ref[...] = pos.astype(i32)
    lane = lax.broadcasted_iota(i32, (128,), 0)
    info = jnp.zeros((128,), f32)
    for k, v in enumerate(starts + tots + nbs):
        info = info + jnp.where(lane == k, v, 0.0)
    binfo_ref[...] = info.astype(i32)


def _bucket(col2d, row2d, ew2d):
    return pl.pallas_call(
        _bucket_body,
        out_shape=[jax.ShapeDtypeStruct((ER, 128), i32),
                   jax.ShapeDtypeStruct((128,), i32),
                   jax.ShapeDtypeStruct((ER, 128), i32)],
    )(col2d, row2d, ew2d)


EPC = EP // NS
WSL = ES2 // NS


def _permute_body(rw_hbm, col_hbm, pos_hbm, rws_hbm, cols_hbm,
                  arrb, posb, pidx, spa):
    c = lax.axis_index("c")
    s = lax.axis_index("s")
    e0 = s * EPC

    @pl.when(c == 0)
    def _():
        pltpu.sync_copy(rw_hbm.at[pl.ds(e0, EPC)], arrb)

    @pl.when(c == 1)
    def _():
        pltpu.sync_copy(col_hbm.at[pl.ds(e0, EPC)], arrb)

    pltpu.sync_copy(pos_hbm.at[pl.ds(e0, EPC)], posb)

    def _bt(k, _):
        off = pl.multiple_of(k * 128, 128)
        for g in range(128 // L):
            pidx[pl.ds(g * L, L)] = posb[pl.ds(off + g * L, L)]
        pltpu.sync_copy(arrb.at[pl.ds(off, 128)], spa.at[pidx])
        return 0
    lax.fori_loop(0, EPC // 128, _bt, 0)
    plsc.subcore_barrier()

    w0 = s * WSL

    @pl.when(c == 0)
    def _():
        pltpu.sync_copy(spa.at[pl.ds(w0, WSL)], rws_hbm.at[pl.ds(w0, WSL)])

    @pl.when(c == 1)
    def _():
        pltpu.sync_copy(spa.at[pl.ds(w0, WSL)], cols_hbm.at[pl.ds(w0, WSL)])


def _permute(rw_p, col_p, pos_p):
    return pl.kernel(
        _permute_body,
        out_type=[jax.ShapeDtypeStruct((ES2,), i32),
                  jax.ShapeDtypeStruct((ES2,), i32)],
        mesh=_mesh(),
        scratch_types=[
            pltpu.VMEM((EPC,), i32),
            pltpu.VMEM((EPC,), i32),
            pltpu.VMEM((128,), i32),
            pltpu.VMEM_SHARED((ES2,), i32),
        ],
    )(rw_p, col_p, pos_p)


def _splat_lane(v, lane):
    idx = jnp.zeros((L,), i32) + lane
    return lax.gather(
        v, idx[:, None],
        dimension_numbers=lax.GatherDimensionNumbers(
            offset_dims=(), collapsed_slice_dims=(0,), start_index_map=(0,)),
        slice_sizes=(1,),
        mode=lax.GatherScatterMode.PROMISE_IN_BOUNDS)


DW = 128
XP = D // DW


def _spmm_body(has_ew, *refs):
    (y_hbm, rws_hbm, cols_hbm, binfo_hbm, parts_hbm,
     rsb, csb, binb, ridx, cidx, gbuf, zb, acc, gsem) = refs

    c = lax.axis_index("c")
    s = lax.axis_index("s")
    wid = c * NS + s
    pltpu.sync_copy(binfo_hbm.at[pl.ds(0, L)], binb)
    bvec = binb[pl.ds(0, L)]

    def _zrow(i, _):
        for u in range(XP):
            for j in range(DW // L):
                zb[i, u, pl.ds(j * L, L)] = _zero16f()
        return 0
    lax.fori_loop(0, 8, _zrow, 0)

    lanes = lax.iota(i32, L)
    dumpv = jnp.zeros((L,), i32) + (SCHUNK + s)

    for ci, span in enumerate(SPANS):
        lo = ci * SCHUNK
        rpw = span // NS
        st = bvec[ci]
        tot = bvec[NCH + ci]
        nbb = bvec[2 * NCH + ci]

        for t in range(rpw // 8):
            pltpu.sync_copy(zb, acc.at[pl.ds(s * rpw + t * 8, 8)])
        plsc.subcore_barrier()

        cnt = (nbb - wid + (NW - 1)) // NW

        def _batch(j, _):
            goff = pl.multiple_of(st + (wid + j * NW) * B, B)
            pltpu.sync_copy(rws_hbm.at[pl.ds(goff, B)], rsb)
            pltpu.sync_copy(cols_hbm.at[pl.ds(goff, B)], csb)
            for g in range(B // L):
                col16 = csb[pl.ds(g * L, L)]
                row16 = lax.shift_right_logical(rsb[pl.ds(g * L, L)], 17)
                valid = (lanes + (goff + g * L)) < (st + tot)
                m = valid & (col16 >= lo) & (col16 < lo + SCHUNK)
                ridx[pl.ds(g * L, L)] = jnp.where(valid, row16, 0)
                cidx[pl.ds(g * L, L)] = jnp.where(m, col16 - lo, dumpv)
            pltpu.async_copy(y_hbm.at[ridx], gbuf, gsem).wait()
            if has_ew:
                def _srow(e, _):
                    o2 = pl.multiple_of((e // L) * L, L)
                    rw16 = rsb[pl.ds(o2, L)]
                    ew16 = (rw16 & (EWQ - 1)).astype(f32) * (1.0 / EWQ)
                    spl = _splat_lane(ew16, e % L)
                    for u in range(XP):
                        for j2 in range(DW // L):
                            gbuf[e, u, pl.ds(j2 * L, L)] = \
                                gbuf[e, u, pl.ds(j2 * L, L)] * spl
                    return 0
                lax.fori_loop(0, B, _srow, 0)
            pltpu.sync_copy(gbuf, acc.at[cidx], add=True)
            return 0
        lax.fori_loop(0, cnt, _batch, 0)
        plsc.subcore_barrier()

        for t in range(rpw // 16):
            r0 = s * rpw + t * 16
            pltpu.sync_copy(acc.at[pl.ds(r0, 16)],
                            parts_hbm.at[c, pl.ds(lo + r0, 16)])
        plsc.subcore_barrier()


def _spmm(y, rws_s, cols_s, binfo, has_ew):
    scratch = [
        pltpu.VMEM((B,), i32),
        pltpu.VMEM((B,), i32),
        pltpu.VMEM((L,), i32),
        pltpu.VMEM((B,), i32),
        pltpu.VMEM((B,), i32),
        pltpu.VMEM((B, XP, DW), f32),
        pltpu.VMEM((8, XP, DW), f32),
        pltpu.VMEM_SHARED((ACC_ROWS, XP, DW), f32),
        pltpu.SemaphoreType.DMA,
    ]
    parts4 = pl.kernel(
        functools.partial(_spmm_body, has_ew),
        out_type=jax.ShapeDtypeStruct((NC, PN, XP, DW), f32),
        mesh=_mesh(),
        scratch_types=scratch,
    )(y.reshape(N, XP, DW), rws_s, cols_s, binfo)
    return parts4.reshape(NC, PN, D)


def _dinv_body(d1_ref, d2_ref, o1_ref, o2_ref):
    o1_ref[...] = lax.rsqrt(d1_ref[0] + d1_ref[1] + 1.0)
    o2_ref[...] = lax.rsqrt(d2_ref[0] + d2_ref[1] + 1.0)


def _dinvs(d1p, d2p):
    return pl.pallas_call(
        _dinv_body,
        out_shape=[jax.ShapeDtypeStruct((PN,), f32),
                   jax.ShapeDtypeStruct((PN,), f32)],
    )(d1p, d2p)


BM = 1000


def _mm_scale_body(x_ref, w_ref, dv_ref, y_ref):
    y_ref[...] = jnp.dot(x_ref[...], w_ref[...],
                         preferred_element_type=f32) * dv_ref[...]


def _mm_scale(x, w, dv):
    return pl.pallas_call(
        _mm_scale_body,
        grid=(N // BM,),
        in_specs=[
            pl.BlockSpec((BM, D), lambda i: (i, 0)),
            pl.BlockSpec((D, D), lambda i: (0, 0)),
            pl.BlockSpec((BM, 1), lambda i: (i, 0)),
        ],
        out_specs=pl.BlockSpec((BM, D), lambda i: (i, 0)),
        out_shape=jax.ShapeDtypeStruct((N, D), f32),
    )(x, w, dv)


def _mid_body(p_ref, y1_ref, dv1_ref, b1_ref, w2_ref, dv2_ref, y2_ref):
    pre = (p_ref[0] + p_ref[1] + y1_ref[...]) * dv1_ref[...] + b1_ref[...]
    h = jnp.maximum(pre, 0.0)
    y2_ref[...] = jnp.dot(h, w2_ref[...], preferred_element_type=f32) * dv2_ref[...]


def _mid(p1, y1, dv1, b1, w2, dv2):
    return pl.pallas_call(
        _mid_body,
        grid=(N // BM,),
        in_specs=[
            pl.BlockSpec((NC, BM, D), lambda i: (0, i, 0)),
            pl.BlockSpec((BM, D), lambda i: (i, 0)),
            pl.BlockSpec((BM, 1), lambda i: (i, 0)),
            pl.BlockSpec((1, D), lambda i: (0, 0)),
            pl.BlockSpec((D, D), lambda i: (0, 0)),
            pl.BlockSpec((BM, 1), lambda i: (i, 0)),
        ],
        out_specs=pl.BlockSpec((BM, D), lambda i: (i, 0)),
        out_shape=jax.ShapeDtypeStruct((N, D), f32),
    )(p1, y1, dv1, b1, w2, dv2)


def _final_body(p_ref, y2_ref, dv2_ref, b2_ref, o_ref):
    o_ref[...] = (p_ref[0] + p_ref[1] + y2_ref[...]) * dv2_ref[...] + b2_ref[...]


def _final(p2, y2, dv2, b2):
    return pl.pallas_call(
        _final_body,
        grid=(N // BM,),
        in_specs=[
            pl.BlockSpec((NC, BM, D), lambda i: (0, i, 0)),
            pl.BlockSpec((BM, D), lambda i: (i, 0)),
            pl.BlockSpec((BM, 1), lambda i: (i, 0)),
            pl.BlockSpec((1, D), lambda i: (0, 0)),
        ],
        out_specs=pl.BlockSpec((BM, D), lambda i: (i, 0)),
        out_shape=jax.ShapeDtypeStruct((N, D), f32),
    )(p2, y2, dv2, b2)


@jax.jit
def kernel(x, edge_index, edge_attr, W1, b1, W2, b2):
    row = edge_index[0]
    col = edge_index[1]

    d1p, d2p = _degrees(col, edge_attr)
    dinv1, dinv2 = _dinvs(d1p, d2p)
    dv1 = dinv1[:N].reshape(N, 1)
    dv2 = dinv2[:N].reshape(N, 1)

    pos2d, binfo, rw2d = _bucket(col.reshape(ER, 128), row.reshape(ER, 128),
                                 edge_attr.reshape(ER, 128))
    npad = EP - E
    pos_p = jnp.concatenate([pos2d.reshape(E),
                             ES + jnp.arange(npad, dtype=i32)])
    rw_p = jnp.concatenate([rw2d.reshape(E), jnp.zeros((npad,), i32)])
    col_p = jnp.concatenate([col, jnp.full((npad,), PN, i32)])
    rws_s, cols_s = _permute(rw_p, col_p, pos_p)

    y1 = _mm_scale(x, W1, dv1)
    p1 = _spmm(y1, rws_s, cols_s, binfo, True)
    y2 = _mid(p1, y1, dv1, b1.reshape(1, D), W2, dv2)
    p2 = _spmm(y2, rws_s, cols_s, binfo, False)
    return _final(p2, y2, dv2, b2.reshape(1, D))


# speedup vs baseline: 4.2030x; 1.1190x over previous
import functools

import jax
import jax.numpy as jnp
import numpy as np
from jax import lax
from jax.experimental import pallas as pl
from jax.experimental.pallas import tpu as pltpu
from jax.experimental.pallas import tpu_sc as plsc

N = 10000
E = 160000
D = 512
PN = 10240

NC, NS, L = 2, 16, 16
NW = NC * NS
EW = E // NW
B = 48
EWPAD = 5120
NBATCH = EWPAD // B
SCHUNK = 2048
NCH = PN // SCHUNK
SPANS = (SCHUNK,) * NCH
ACC_ROWS = SCHUNK + NS

EP = EWPAD * NW
ES = E + NCH * B * 2
ES2 = -(-(ES + (EP - E)) // 2048) * 2048
ER = E // 128

def _mesh():
    return plsc.VectorSubcoreMesh(core_axis_name="c", subcore_axis_name="s")

f32 = jnp.float32
i32 = jnp.int32


def _zero16f():
    return jnp.zeros((L,), f32)


def _zero16i():
    return jnp.zeros((L,), i32)


def _deg_body(col_hbm, ew_hbm, d1_hbm, d2_hbm,
              colb, ewb, idxb, idxt, valt, onesb, zb, d1acc, d2acc):
    c = lax.axis_index("c")
    s = lax.axis_index("s")
    wid = c * NS + s
    e0 = wid * EW
    pltpu.sync_copy(col_hbm.at[pl.ds(e0, EW)], colb.at[pl.ds(0, EW)])
    pltpu.sync_copy(ew_hbm.at[pl.ds(e0, EW)], ewb.at[pl.ds(0, EW)])

    def _fill(i, _):
        off = pl.multiple_of(i * L, L)
        zb[pl.ds(off, L)] = _zero16f()
        return 0
    lax.fori_loop(0, 640 // L, _fill, 0)
    for g in range(128 // L):
        onesb[pl.ds(g * L, L)] = jnp.ones((L,), f32)

    pltpu.sync_copy(zb, d1acc.at[pl.ds(s * 640, 640)])
    pltpu.sync_copy(zb, d2acc.at[pl.ds(s * 640, 640)])
    plsc.subcore_barrier()

    def _batch(k, _):
        off = pl.multiple_of(k * 128, 128)
        for g in range(128 // L):
            idxb[pl.ds(g * L, L)] = colb[pl.ds(off + g * L, L)]
        pltpu.sync_copy(ewb.at[pl.ds(off, 128)], d1acc.at[idxb], add=True)
        pltpu.sync_copy(onesb, d2acc.at[idxb], add=True)
        return 0
    lax.fori_loop(0, EW // 128, _batch, 0)

    toff = (EW // 128) * 128
    valid = lax.iota(i32, L) < (EW - toff)
    col16 = colb[pl.ds(toff, L)]
    ew16 = ewb[pl.ds(toff, L)]
    idxt[pl.ds(0, L)] = jnp.where(valid, col16, 0)
    valt[pl.ds(0, L)] = jnp.where(valid, ew16, 0.0)
    pltpu.sync_copy(valt, d1acc.at[idxt], add=True)
    valt[pl.ds(0, L)] = jnp.where(valid, jnp.ones((L,), f32), 0.0)
    pltpu.sync_copy(valt, d2acc.at[idxt], add=True)

    plsc.subcore_barrier()

    @pl.when(s == 0)
    def _():
        pltpu.sync_copy(d1acc, d1_hbm.at[c])
        pltpu.sync_copy(d2acc, d2_hbm.at[c])


def _degrees(col, ew):
    return pl.kernel(
        _deg_body,
        out_type=[jax.ShapeDtypeStruct((NC, PN), f32),
                  jax.ShapeDtypeStruct((NC, PN), f32)],
        mesh=_mesh(),
        scratch_types=[
            pltpu.VMEM((EWPAD,), i32),
            pltpu.VMEM((EWPAD,), f32),
            pltpu.VMEM((128,), i32),
            pltpu.VMEM((L,), i32),
            pltpu.VMEM((L,), f32),
            pltpu.VMEM((128,), f32),
            pltpu.VMEM((640,), f32),
            pltpu.VMEM_SHARED((PN,), f32),
            pltpu.VMEM_SHARED((PN,), f32),
        ],
    )(col, ew)


EWQ = 131072


def _bucket_body(col_ref, row_ref, ew_ref, pos_ref, binfo_ref, rw_ref):
    cols = col_ref[...]
    rows = row_ref[...]
    ews = ew_ref[...]
    rw_ref[...] = rows * EWQ + jnp.floor(ews * EWQ).astype(i32)
    q = lax.shift_right_logical(cols, 11)
    jrow = lax.broadcasted_iota(i32, (128, 128), 0)
    jcol = lax.broadcasted_iota(i32, (128, 128), 1)
    su128 = (jrow < jcol).astype(f32)
    irow = lax.broadcasted_iota(i32, (ER, ER), 0)
    icol = lax.broadcasted_iota(i32, (ER, ER), 1)
    suer = (irow < icol).astype(f32)

    pos = jnp.zeros(cols.shape, f32)
    start = 0.0
    starts, tots, nbs = [], [], []
    for qq in range(NCH):
        m = (q == qq).astype(f32)
        within = jnp.dot(m, su128, preferred_element_type=f32)
        rs = jnp.sum(m, axis=1)
        rp = jnp.dot(rs.reshape(1, ER), suer,
                     preferred_element_type=f32)
        posq = rp.reshape(ER, 1) + within
        tot = jnp.sum(rs)
        pos = pos + m * (posq + start)
        starts.append(start)
        tots.append(tot)
        nbs.append(jnp.ceil(tot / B))
        start = start + jnp.ceil(tot / B) * B
    pos_ref[...] = pos.astype(i32)
    lane = lax.broadcasted_iota(i32, (128,), 0)
    info = jnp.zeros((128,), f32)
    for k, v in enumerate(starts + tots + nbs):
        info = info + jnp.where(lane == k, v, 0.0)
    binfo_ref[...] = info.astype(i32)


def _bucket(col2d, row2d, ew2d):
    return pl.pallas_call(
        _bucket_body,
        out_shape=[jax.ShapeDtypeStruct((ER, 128), i32),
                   jax.ShapeDtypeStruct((128,), i32),
                   jax.ShapeDtypeStruct((ER, 128), i32)],
    )(col2d, row2d, ew2d)


EPC = EP // NS
WSL = ES2 // NS


def _permute_body(rw_hbm, col_hbm, pos_hbm, rws_hbm, cols_hbm,
                  arrb, posb, pidx, spa):
    c = lax.axis_index("c")
    s = lax.axis_index("s")
    e0 = s * EPC

    @pl.when(c == 0)
    def _():
        pltpu.sync_copy(rw_hbm.at[pl.ds(e0, EPC)], arrb)

    @pl.when(c == 1)
    def _():
        pltpu.sync_copy(col_hbm.at[pl.ds(e0, EPC)], arrb)

    pltpu.sync_copy(pos_hbm.at[pl.ds(e0, EPC)], posb)

    def _bt(k, _):
        off = pl.multiple_of(k * 128, 128)
        for g in range(128 // L):
            pidx[pl.ds(g * L, L)] = posb[pl.ds(off + g * L, L)]
        pltpu.sync_copy(arrb.at[pl.ds(off, 128)], spa.at[pidx])
        return 0
    lax.fori_loop(0, EPC // 128, _bt, 0)
    plsc.subcore_barrier()

    w0 = s * WSL

    @pl.when(c == 0)
    def _():
        pltpu.sync_copy(spa.at[pl.ds(w0, WSL)], rws_hbm.at[pl.ds(w0, WSL)])

    @pl.when(c == 1)
    def _():
        pltpu.sync_copy(spa.at[pl.ds(w0, WSL)], cols_hbm.at[pl.ds(w0, WSL)])


def _permute(rw_p, col_p, pos_p):
    return pl.kernel(
        _permute_body,
        out_type=[jax.ShapeDtypeStruct((ES2,), i32),
                  jax.ShapeDtypeStruct((ES2,), i32)],
        mesh=_mesh(),
        scratch_types=[
            pltpu.VMEM((EPC,), i32),
            pltpu.VMEM((EPC,), i32),
            pltpu.VMEM((128,), i32),
            pltpu.VMEM_SHARED((ES2,), i32),
        ],
    )(rw_p, col_p, pos_p)


def _splat_lane(v, lane):
    idx = jnp.zeros((L,), i32) + lane
    return lax.gather(
        v, idx[:, None],
        dimension_numbers=lax.GatherDimensionNumbers(
            offset_dims=(), collapsed_slice_dims=(0,), start_index_map=(0,)),
        slice_sizes=(1,),
        mode=lax.GatherScatterMode.PROMISE_IN_BOUNDS)


DW = 128
XP = D // DW


def _spmm_body(has_ew, *refs):
    (y_hbm, rws_hbm, cols_hbm, binfo_hbm, parts_hbm,
     rsb, csb, rsb2, csb2, binb, ridx, cidx, ridx2, cidx2,
     gbuf, gbuf2, zb, acc, gsem, ssem, ssem2) = refs

    c = lax.axis_index("c")
    s = lax.axis_index("s")
    wid = c * NS + s
    pltpu.sync_copy(binfo_hbm.at[pl.ds(0, L)], binb)
    bvec = binb[pl.ds(0, L)]

    def _zrow(i, _):
        for u in range(XP):
            for j in range(DW // L):
                zb[i, u, pl.ds(j * L, L)] = _zero16f()
        return 0
    lax.fori_loop(0, 8, _zrow, 0)

    lanes = lax.iota(i32, L)
    dumpv = jnp.zeros((L,), i32) + (SCHUNK + s)

    for ci, span in enumerate(SPANS):
        lo = ci * SCHUNK
        rpw = span // NS
        st = bvec[ci]
        tot = bvec[NCH + ci]
        nbb = bvec[2 * NCH + ci]

        for t in range(rpw // 8):
            pltpu.sync_copy(zb, acc.at[pl.ds(s * rpw + t * 8, 8)])
        plsc.subcore_barrier()

        cnt = (nbb - wid + (NW - 1)) // NW
        cnt2 = (cnt + 1) // 2

        def _batch2(mi, _):
            for half, (gb, rb, cb, rx, cx, ss) in enumerate(
                    ((gbuf, rsb, csb, ridx, cidx, ssem),
                     (gbuf2, rsb2, csb2, ridx2, cidx2, ssem2))):
                j = mi * 2 + half

                @pl.when(j < cnt)
                def _():
                    goff = pl.multiple_of(st + (wid + j * NW) * B, 8)

                    @pl.when(mi > 0)
                    def _():
                        pltpu.make_async_copy(gb, acc.at[cx], ss).wait()

                    pltpu.sync_copy(rws_hbm.at[pl.ds(goff, B)], rb)
                    pltpu.sync_copy(cols_hbm.at[pl.ds(goff, B)], cb)
                    for g in range(B // L):
                        col16 = cb[pl.ds(g * L, L)]
                        row16 = lax.shift_right_logical(rb[pl.ds(g * L, L)], 17)
                        valid = (lanes + (goff + g * L)) < (st + tot)
                        m = valid & (col16 >= lo) & (col16 < lo + SCHUNK)
                        rx[pl.ds(g * L, L)] = jnp.where(valid, row16, 0)
                        cx[pl.ds(g * L, L)] = jnp.where(m, col16 - lo, dumpv)
                    pltpu.async_copy(y_hbm.at[rx], gb, gsem).wait()
                    if has_ew:
                        def _srow(e, _):
                            o2 = pl.multiple_of((e // L) * L, L)
                            rw16 = rb[pl.ds(o2, L)]
                            ew16 = (rw16 & (EWQ - 1)).astype(f32) * (1.0 / EWQ)
                            spl = _splat_lane(ew16, e % L)
                            for u in range(XP):
                                for j2 in range(DW // L):
                                    gb[e, u, pl.ds(j2 * L, L)] = \
                                        gb[e, u, pl.ds(j2 * L, L)] * spl
                            return 0
                        lax.fori_loop(0, B, _srow, 0)
                    pltpu.async_copy(gb, acc.at[cx], ss, add=True)
            return 0
        lax.fori_loop(0, cnt2, _batch2, 0)

        @pl.when(cnt > 0)
        def _():
            pltpu.make_async_copy(gbuf, acc.at[cidx], ssem).wait()

        @pl.when(cnt > 1)
        def _():
            pltpu.make_async_copy(gbuf2, acc.at[cidx2], ssem2).wait()

        plsc.subcore_barrier()

        for t in range(rpw // 16):
            r0 = s * rpw + t * 16
            pltpu.sync_copy(acc.at[pl.ds(r0, 16)],
                            parts_hbm.at[c, pl.ds(lo + r0, 16)])
        plsc.subcore_barrier()


def _spmm(y, rws_s, cols_s, binfo, has_ew):
    scratch = [
        pltpu.VMEM((B,), i32),
        pltpu.VMEM((B,), i32),
        pltpu.VMEM((B,), i32),
        pltpu.VMEM((B,), i32),
        pltpu.VMEM((L,), i32),
        pltpu.VMEM((B,), i32),
        pltpu.VMEM((B,), i32),
        pltpu.VMEM((B,), i32),
        pltpu.VMEM((B,), i32),
        pltpu.VMEM((B, XP, DW), f32),
        pltpu.VMEM((B, XP, DW), f32),
        pltpu.VMEM((8, XP, DW), f32),
        pltpu.VMEM_SHARED((ACC_ROWS, XP, DW), f32),
        pltpu.SemaphoreType.DMA,
        pltpu.SemaphoreType.DMA,
        pltpu.SemaphoreType.DMA,
    ]
    parts4 = pl.kernel(
        functools.partial(_spmm_body, has_ew),
        out_type=jax.ShapeDtypeStruct((NC, PN, XP, DW), f32),
        mesh=_mesh(),
        scratch_types=scratch,
    )(y.reshape(N, XP, DW), rws_s, cols_s, binfo)
    return parts4.reshape(NC, PN, D)


def _dinv_body(d1_ref, d2_ref, o1_ref, o2_ref):
    o1_ref[...] = lax.rsqrt(d1_ref[0] + d1_ref[1] + 1.0)
    o2_ref[...] = lax.rsqrt(d2_ref[0] + d2_ref[1] + 1.0)


def _dinvs(d1p, d2p):
    return pl.pallas_call(
        _dinv_body,
        out_shape=[jax.ShapeDtypeStruct((PN,), f32),
                   jax.ShapeDtypeStruct((PN,), f32)],
    )(d1p, d2p)


BM = 1000


def _mm_scale_body(x_ref, w_ref, dv_ref, y_ref):
    y_ref[...] = jnp.dot(x_ref[...], w_ref[...],
                         preferred_element_type=f32) * dv_ref[...]


def _mm_scale(x, w, dv):
    return pl.pallas_call(
        _mm_scale_body,
        grid=(N // BM,),
        in_specs=[
            pl.BlockSpec((BM, D), lambda i: (i, 0)),
            pl.BlockSpec((D, D), lambda i: (0, 0)),
            pl.BlockSpec((BM, 1), lambda i: (i, 0)),
        ],
        out_specs=pl.BlockSpec((BM, D), lambda i: (i, 0)),
        out_shape=jax.ShapeDtypeStruct((N, D), f32),
    )(x, w, dv)


def _mid_body(p_ref, y1_ref, dv1_ref, b1_ref, w2_ref, dv2_ref, y2_ref):
    pre = (p_ref[0] + p_ref[1] + y1_ref[...]) * dv1_ref[...] + b1_ref[...]
    h = jnp.maximum(pre, 0.0)
    y2_ref[...] = jnp.dot(h, w2_ref[...], preferred_element_type=f32) * dv2_ref[...]


def _mid(p1, y1, dv1, b1, w2, dv2):
    return pl.pallas_call(
        _mid_body,
        grid=(N // BM,),
        in_specs=[
            pl.BlockSpec((NC, BM, D), lambda i: (0, i, 0)),
            pl.BlockSpec((BM, D), lambda i: (i, 0)),
            pl.BlockSpec((BM, 1), lambda i: (i, 0)),
            pl.BlockSpec((1, D), lambda i: (0, 0)),
            pl.BlockSpec((D, D), lambda i: (0, 0)),
            pl.BlockSpec((BM, 1), lambda i: (i, 0)),
        ],
        out_specs=pl.BlockSpec((BM, D), lambda i: (i, 0)),
        out_shape=jax.ShapeDtypeStruct((N, D), f32),
    )(p1, y1, dv1, b1, w2, dv2)


def _final_body(p_ref, y2_ref, dv2_ref, b2_ref, o_ref):
    o_ref[...] = (p_ref[0] + p_ref[1] + y2_ref[...]) * dv2_ref[...] + b2_ref[...]


def _final(p2, y2, dv2, b2):
    return pl.pallas_call(
        _final_body,
        grid=(N // BM,),
        in_specs=[
            pl.BlockSpec((NC, BM, D), lambda i: (0, i, 0)),
            pl.BlockSpec((BM, D), lambda i: (i, 0)),
            pl.BlockSpec((BM, 1), lambda i: (i, 0)),
            pl.BlockSpec((1, D), lambda i: (0, 0)),
        ],
        out_specs=pl.BlockSpec((BM, D), lambda i: (i, 0)),
        out_shape=jax.ShapeDtypeStruct((N, D), f32),
    )(p2, y2, dv2, b2)


@jax.jit
def kernel(x, edge_index, edge_attr, W1, b1, W2, b2):
    row = edge_index[0]
    col = edge_index[1]

    d1p, d2p = _degrees(col, edge_attr)
    dinv1, dinv2 = _dinvs(d1p, d2p)
    dv1 = dinv1[:N].reshape(N, 1)
    dv2 = dinv2[:N].reshape(N, 1)

    pos2d, binfo, rw2d = _bucket(col.reshape(ER, 128), row.reshape(ER, 128),
                                 edge_attr.reshape(ER, 128))
    npad = EP - E
    pos_p = jnp.concatenate([pos2d.reshape(E),
                             ES + jnp.arange(npad, dtype=i32)])
    rw_p = jnp.concatenate([rw2d.reshape(E), jnp.zeros((npad,), i32)])
    col_p = jnp.concatenate([col, jnp.full((npad,), PN, i32)])
    rws_s, cols_s = _permute(rw_p, col_p, pos_p)

    y1 = _mm_scale(x, W1, dv1)
    p1 = _spmm(y1, rws_s, cols_s, binfo, True)
    y2 = _mid(p1, y1, dv1, b1.reshape(1, D), W2, dv2)
    p2 = _spmm(y2, rws_s, cols_s, binfo, False)
    return _final(p2, y2, dv2, b2.reshape(1, D))


# speedup vs baseline: 5.0150x; 1.1932x over previous
import functools

import jax
import jax.numpy as jnp
import numpy as np
from jax import lax
from jax.experimental import pallas as pl
from jax.experimental.pallas import tpu as pltpu
from jax.experimental.pallas import tpu_sc as plsc

N = 10000
E = 160000
D = 512
PN = 10240

NC, NS, L = 2, 16, 16
NW = NC * NS
EW = E // NW
B = 48
EWPAD = 5120
NBATCH = EWPAD // B
SCHUNK = 2048
NCH = PN // SCHUNK
SPANS = (SCHUNK,) * NCH
ACC_ROWS = SCHUNK + NS

EP = EWPAD * NW
ES = E + NCH * B * 2
ES2 = -(-(ES + (EP - E)) // 2048) * 2048
ER = E // 128

def _mesh():
    return plsc.VectorSubcoreMesh(core_axis_name="c", subcore_axis_name="s")

f32 = jnp.float32
i32 = jnp.int32


def _zero16f():
    return jnp.zeros((L,), f32)


def _zero16i():
    return jnp.zeros((L,), i32)


def _deg_body(col_hbm, ew_hbm, d1_hbm, d2_hbm,
              colb, ewb, idxb, idxt, valt, onesb, zb, d1acc, d2acc):
    c = lax.axis_index("c")
    s = lax.axis_index("s")
    wid = c * NS + s
    e0 = wid * EW
    pltpu.sync_copy(col_hbm.at[pl.ds(e0, EW)], colb.at[pl.ds(0, EW)])
    pltpu.sync_copy(ew_hbm.at[pl.ds(e0, EW)], ewb.at[pl.ds(0, EW)])

    def _fill(i, _):
        off = pl.multiple_of(i * L, L)
        zb[pl.ds(off, L)] = _zero16f()
        return 0
    lax.fori_loop(0, 640 // L, _fill, 0)
    for g in range(128 // L):
        onesb[pl.ds(g * L, L)] = jnp.ones((L,), f32)

    pltpu.sync_copy(zb, d1acc.at[pl.ds(s * 640, 640)])
    pltpu.sync_copy(zb, d2acc.at[pl.ds(s * 640, 640)])
    plsc.subcore_barrier()

    def _batch(k, _):
        off = pl.multiple_of(k * 128, 128)
        for g in range(128 // L):
            idxb[pl.ds(g * L, L)] = colb[pl.ds(off + g * L, L)]
        pltpu.sync_copy(ewb.at[pl.ds(off, 128)], d1acc.at[idxb], add=True)
        pltpu.sync_copy(onesb, d2acc.at[idxb], add=True)
        return 0
    lax.fori_loop(0, EW // 128, _batch, 0)

    toff = (EW // 128) * 128
    valid = lax.iota(i32, L) < (EW - toff)
    col16 = colb[pl.ds(toff, L)]
    ew16 = ewb[pl.ds(toff, L)]
    idxt[pl.ds(0, L)] = jnp.where(valid, col16, 0)
    valt[pl.ds(0, L)] = jnp.where(valid, ew16, 0.0)
    pltpu.sync_copy(valt, d1acc.at[idxt], add=True)
    valt[pl.ds(0, L)] = jnp.where(valid, jnp.ones((L,), f32), 0.0)
    pltpu.sync_copy(valt, d2acc.at[idxt], add=True)

    plsc.subcore_barrier()

    @pl.when(s == 0)
    def _():
        pltpu.sync_copy(d1acc, d1_hbm.at[c])
        pltpu.sync_copy(d2acc, d2_hbm.at[c])


def _degrees(col, ew):
    return pl.kernel(
        _deg_body,
        out_type=[jax.ShapeDtypeStruct((NC, PN), f32),
                  jax.ShapeDtypeStruct((NC, PN), f32)],
        mesh=_mesh(),
        scratch_types=[
            pltpu.VMEM((EWPAD,), i32),
            pltpu.VMEM((EWPAD,), f32),
            pltpu.VMEM((128,), i32),
            pltpu.VMEM((L,), i32),
            pltpu.VMEM((L,), f32),
            pltpu.VMEM((128,), f32),
            pltpu.VMEM((640,), f32),
            pltpu.VMEM_SHARED((PN,), f32),
            pltpu.VMEM_SHARED((PN,), f32),
        ],
    )(col, ew)


EWQ = 131072


def _bucket_body(col_ref, row_ref, ew_ref, pos_ref, binfo_ref, rw_ref):
    cols = col_ref[...]
    rows = row_ref[...]
    ews = ew_ref[...]
    rw_ref[...] = rows * EWQ + jnp.floor(ews * EWQ).astype(i32)
    q = lax.shift_right_logical(cols, 11)
    jrow = lax.broadcasted_iota(i32, (128, 128), 0)
    jcol = lax.broadcasted_iota(i32, (128, 128), 1)
    su128 = (jrow < jcol).astype(f32)
    irow = lax.broadcasted_iota(i32, (ER, ER), 0)
    icol = lax.broadcasted_iota(i32, (ER, ER), 1)
    suer = (irow < icol).astype(f32)

    pos = jnp.zeros(cols.shape, f32)
    start = 0.0
    starts, tots, nbs = [], [], []
    for qq in range(NCH):
        m = (q == qq).astype(f32)
        within = jnp.dot(m, su128, preferred_element_type=f32)
        rs = jnp.sum(m, axis=1)
        rp = jnp.dot(rs.reshape(1, ER), suer,
                     preferred_element_type=f32)
        posq = rp.reshape(ER, 1) + within
        tot = jnp.sum(rs)
        pos = pos + m * (posq + start)
        starts.append(start)
        tots.append(tot)
        nbs.append(jnp.ceil(tot / B))
        start = start + jnp.ceil(tot / B) * B
    pos_ref[...] = pos.astype(i32)
    lane = lax.broadcasted_iota(i32, (128,), 0)
    info = jnp.zeros((128,), f32)
    for k, v in enumerate(starts + tots + nbs):
        info = info + jnp.where(lane == k, v, 0.0)
    binfo_ref[...] = info.astype(i32)


def _bucket(col2d, row2d, ew2d):
    return pl.pallas_call(
        _bucket_body,
        out_shape=[jax.ShapeDtypeStruct((ER, 128), i32),
                   jax.ShapeDtypeStruct((128,), i32),
                   jax.ShapeDtypeStruct((ER, 128), i32)],
    )(col2d, row2d, ew2d)


EPC = EP // NS
WSL = ES2 // NS


def _permute_body(rw_hbm, col_hbm, pos_hbm, rws_hbm, cols_hbm,
                  arrb, posb, pidx, spa):
    c = lax.axis_index("c")
    s = lax.axis_index("s")
    e0 = s * EPC

    @pl.when(c == 0)
    def _():
        pltpu.sync_copy(rw_hbm.at[pl.ds(e0, EPC)], arrb)

    @pl.when(c == 1)
    def _():
        pltpu.sync_copy(col_hbm.at[pl.ds(e0, EPC)], arrb)

    pltpu.sync_copy(pos_hbm.at[pl.ds(e0, EPC)], posb)

    def _bt(k, _):
        off = pl.multiple_of(k * 128, 128)
        for g in range(128 // L):
            pidx[pl.ds(g * L, L)] = posb[pl.ds(off + g * L, L)]
        pltpu.sync_copy(arrb.at[pl.ds(off, 128)], spa.at[pidx])
        return 0
    lax.fori_loop(0, EPC // 128, _bt, 0)
    plsc.subcore_barrier()

    w0 = s * WSL

    @pl.when(c == 0)
    def _():
        pltpu.sync_copy(spa.at[pl.ds(w0, WSL)], rws_hbm.at[pl.ds(w0, WSL)])

    @pl.when(c == 1)
    def _():
        pltpu.sync_copy(spa.at[pl.ds(w0, WSL)], cols_hbm.at[pl.ds(w0, WSL)])


def _permute(rw_p, col_p, pos_p):
    return pl.kernel(
        _permute_body,
        out_type=[jax.ShapeDtypeStruct((ES2,), i32),
                  jax.ShapeDtypeStruct((ES2,), i32)],
        mesh=_mesh(),
        scratch_types=[
            pltpu.VMEM((EPC,), i32),
            pltpu.VMEM((EPC,), i32),
            pltpu.VMEM((128,), i32),
            pltpu.VMEM_SHARED((ES2,), i32),
        ],
    )(rw_p, col_p, pos_p)


def _splat_lane(v, lane):
    idx = jnp.zeros((L,), i32) + lane
    return lax.gather(
        v, idx[:, None],
        dimension_numbers=lax.GatherDimensionNumbers(
            offset_dims=(), collapsed_slice_dims=(0,), start_index_map=(0,)),
        slice_sizes=(1,),
        mode=lax.GatherScatterMode.PROMISE_IN_BOUNDS)


DW = 128
XP = D // DW


def _spmm_body(has_ew, *refs):
    (y_hbm, rws_hbm, cols_hbm, binfo_hbm, parts_hbm,
     rsb, csb, rsb2, csb2, binb, ridx, cidx, ridx2, cidx2,
     gbuf, gbuf2, zb, acc, gsem, gsem2, ssem, ssem2) = refs

    c = lax.axis_index("c")
    s = lax.axis_index("s")
    wid = c * NS + s
    pltpu.sync_copy(binfo_hbm.at[pl.ds(0, L)], binb)
    bvec = binb[pl.ds(0, L)]

    def _zrow(i, _):
        for u in range(XP):
            for j in range(DW // L):
                zb[i, u, pl.ds(j * L, L)] = _zero16f()
        return 0
    lax.fori_loop(0, 8, _zrow, 0)

    lanes = lax.iota(i32, L)
    dumpv = jnp.zeros((L,), i32) + (SCHUNK + s)

    for ci, span in enumerate(SPANS):
        lo = ci * SCHUNK
        rpw = span // NS
        st = bvec[ci]
        tot = bvec[NCH + ci]
        nbb = bvec[2 * NCH + ci]

        for t in range(rpw // 8):
            pltpu.sync_copy(zb, acc.at[pl.ds(s * rpw + t * 8, 8)])
        plsc.subcore_barrier()

        cnt = (nbb - wid + (NW - 1)) // NW
        cnt2 = (cnt + 1) // 2

        halves = ((gbuf, rsb, csb, ridx, cidx, ssem, gsem),
                  (gbuf2, rsb2, csb2, ridx2, cidx2, ssem2, gsem2))

        def _batch2(mi, _):
            for half, (gb, rb, cb, rx, cx, ss, gs) in enumerate(halves):
                j = mi * 2 + half

                @pl.when(j < cnt)
                def _():
                    goff = pl.multiple_of(st + (wid + j * NW) * B, 8)

                    @pl.when(mi > 0)
                    def _():
                        pltpu.make_async_copy(gb, acc.at[cx], ss).wait()

                    pltpu.sync_copy(rws_hbm.at[pl.ds(goff, B)], rb)
                    pltpu.sync_copy(cols_hbm.at[pl.ds(goff, B)], cb)
                    for g in range(B // L):
                        col16 = cb[pl.ds(g * L, L)]
                        row16 = lax.shift_right_logical(rb[pl.ds(g * L, L)], 17)
                        valid = (lanes + (goff + g * L)) < (st + tot)
                        m = valid & (col16 >= lo) & (col16 < lo + SCHUNK)
                        rx[pl.ds(g * L, L)] = jnp.where(valid, row16, 0)
                        cx[pl.ds(g * L, L)] = jnp.where(m, col16 - lo, dumpv)
                    pltpu.async_copy(y_hbm.at[rx], gb, gs)

            for half, (gb, rb, cb, rx, cx, ss, gs) in enumerate(halves):
                j = mi * 2 + half

                @pl.when(j < cnt)
                def _():
                    pltpu.make_async_copy(y_hbm.at[rx], gb, gs).wait()
                    if has_ew:
                        def _srow(e, _):
                            o2 = pl.multiple_of((e // L) * L, L)
                            rw16 = rb[pl.ds(o2, L)]
                            ew16 = (rw16 & (EWQ - 1)).astype(f32) * (1.0 / EWQ)
                            spl = _splat_lane(ew16, e % L)
                            for u in range(XP):
                                for j2 in range(DW // L):
                                    gb[e, u, pl.ds(j2 * L, L)] = \
                                        gb[e, u, pl.ds(j2 * L, L)] * spl
                            return 0
                        lax.fori_loop(0, B, _srow, 0)
                    pltpu.async_copy(gb, acc.at[cx], ss, add=True)
            return 0
        lax.fori_loop(0, cnt2, _batch2, 0)

        @pl.when(cnt > 0)
        def _():
            pltpu.make_async_copy(gbuf, acc.at[cidx], ssem).wait()

        @pl.when(cnt > 1)
        def _():
            pltpu.make_async_copy(gbuf2, acc.at[cidx2], ssem2).wait()

        plsc.subcore_barrier()

        for t in range(rpw // 16):
            r0 = s * rpw + t * 16
            pltpu.sync_copy(acc.at[pl.ds(r0, 16)],
                            parts_hbm.at[c, pl.ds(lo + r0, 16)])
        plsc.subcore_barrier()


def _spmm(y, rws_s, cols_s, binfo, has_ew):
    scratch = [
        pltpu.VMEM((B,), i32),
        pltpu.VMEM((B,), i32),
        pltpu.VMEM((B,), i32),
        pltpu.VMEM((B,), i32),
        pltpu.VMEM((L,), i32),
        pltpu.VMEM((B,), i32),
        pltpu.VMEM((B,), i32),
        pltpu.VMEM((B,), i32),
        pltpu.VMEM((B,), i32),
        pltpu.VMEM((B, XP, DW), f32),
        pltpu.VMEM((B, XP, DW), f32),
        pltpu.VMEM((8, XP, DW), f32),
        pltpu.VMEM_SHARED((ACC_ROWS, XP, DW), f32),
        pltpu.SemaphoreType.DMA,
        pltpu.SemaphoreType.DMA,
        pltpu.SemaphoreType.DMA,
        pltpu.SemaphoreType.DMA,
    ]
    parts4 = pl.kernel(
        functools.partial(_spmm_body, has_ew),
        out_type=jax.ShapeDtypeStruct((NC, PN, XP, DW), f32),
        mesh=_mesh(),
        scratch_types=scratch,
    )(y.reshape(N, XP, DW), rws_s, cols_s, binfo)
    return parts4.reshape(NC, PN, D)


def _dinv_body(d1_ref, d2_ref, o1_ref, o2_ref):
    o1_ref[...] = lax.rsqrt(d1_ref[0] + d1_ref[1] + 1.0)
    o2_ref[...] = lax.rsqrt(d2_ref[0] + d2_ref[1] + 1.0)


def _dinvs(d1p, d2p):
    return pl.pallas_call(
        _dinv_body,
        out_shape=[jax.ShapeDtypeStruct((PN,), f32),
                   jax.ShapeDtypeStruct((PN,), f32)],
    )(d1p, d2p)


BM = 1000


def _mm_scale_body(x_ref, w_ref, dv_ref, y_ref):
    y_ref[...] = jnp.dot(x_ref[...], w_ref[...],
                         preferred_element_type=f32) * dv_ref[...]


def _mm_scale(x, w, dv):
    return pl.pallas_call(
        _mm_scale_body,
        grid=(N // BM,),
        in_specs=[
            pl.BlockSpec((BM, D), lambda i: (i, 0)),
            pl.BlockSpec((D, D), lambda i: (0, 0)),
            pl.BlockSpec((BM, 1), lambda i: (i, 0)),
        ],
        out_specs=pl.BlockSpec((BM, D), lambda i: (i, 0)),
        out_shape=jax.ShapeDtypeStruct((N, D), f32),
    )(x, w, dv)


def _mid_body(p_ref, y1_ref, dv1_ref, b1_ref, w2_ref, dv2_ref, y2_ref):
    pre = (p_ref[0] + p_ref[1] + y1_ref[...]) * dv1_ref[...] + b1_ref[...]
    h = jnp.maximum(pre, 0.0)
    y2_ref[...] = jnp.dot(h, w2_ref[...], preferred_element_type=f32) * dv2_ref[...]


def _mid(p1, y1, dv1, b1, w2, dv2):
    return pl.pallas_call(
        _mid_body,
        grid=(N // BM,),
        in_specs=[
            pl.BlockSpec((NC, BM, D), lambda i: (0, i, 0)),
            pl.BlockSpec((BM, D), lambda i: (i, 0)),
            pl.BlockSpec((BM, 1), lambda i: (i, 0)),
            pl.BlockSpec((1, D), lambda i: (0, 0)),
            pl.BlockSpec((D, D), lambda i: (0, 0)),
            pl.BlockSpec((BM, 1), lambda i: (i, 0)),
        ],
        out_specs=pl.BlockSpec((BM, D), lambda i: (i, 0)),
        out_shape=jax.ShapeDtypeStruct((N, D), f32),
    )(p1, y1, dv1, b1, w2, dv2)


def _final_body(p_ref, y2_ref, dv2_ref, b2_ref, o_ref):
    o_ref[...] = (p_ref[0] + p_ref[1] + y2_ref[...]) * dv2_ref[...] + b2_ref[...]


def _final(p2, y2, dv2, b2):
    return pl.pallas_call(
        _final_body,
        grid=(N // BM,),
        in_specs=[
            pl.BlockSpec((NC, BM, D), lambda i: (0, i, 0)),
            pl.BlockSpec((BM, D), lambda i: (i, 0)),
            pl.BlockSpec((BM, 1), lambda i: (i, 0)),
            pl.BlockSpec((1, D), lambda i: (0, 0)),
        ],
        out_specs=pl.BlockSpec((BM, D), lambda i: (i, 0)),
        out_shape=jax.ShapeDtypeStruct((N, D), f32),
    )(p2, y2, dv2, b2)


@jax.jit
def kernel(x, edge_index, edge_attr, W1, b1, W2, b2):
    row = edge_index[0]
    col = edge_index[1]

    d1p, d2p = _degrees(col, edge_attr)
    dinv1, dinv2 = _dinvs(d1p, d2p)
    dv1 = dinv1[:N].reshape(N, 1)
    dv2 = dinv2[:N].reshape(N, 1)

    pos2d, binfo, rw2d = _bucket(col.reshape(ER, 128), row.reshape(ER, 128),
                                 edge_attr.reshape(ER, 128))
    npad = EP - E
    pos_p = jnp.concatenate([pos2d.reshape(E),
                             ES + jnp.arange(npad, dtype=i32)])
    rw_p = jnp.concatenate([rw2d.reshape(E), jnp.zeros((npad,), i32)])
    col_p = jnp.concatenate([col, jnp.full((npad,), PN, i32)])
    rws_s, cols_s = _permute(rw_p, col_p, pos_p)

    y1 = _mm_scale(x, W1, dv1)
    p1 = _spmm(y1, rws_s, cols_s, binfo, True)
    y2 = _mid(p1, y1, dv1, b1.reshape(1, D), W2, dv2)
    p2 = _spmm(y2, rws_s, cols_s, binfo, False)
    return _final(p2, y2, dv2, b2.reshape(1, D))
